# baseline (speedup 1.0000x reference)
import functools

import jax
import jax.numpy as jnp
import numpy as np
from jax import lax
from jax.experimental import pallas as pl
from jax.experimental.pallas import tpu as pltpu
from jax.experimental.pallas import tpu_sc as plsc

EPS = 1e-06
L = 16
NC = 2
NS = 16
NW = NC * NS


def _sqrt16(x):
    i = plsc.bitcast(x, jnp.int32)
    y = plsc.bitcast((i >> 1) + jnp.int32(0x1FBD1DF5), jnp.float32)
    y = 0.5 * (y + x / y)
    y = 0.5 * (y + x / y)
    return y


def _silu16(x):
    return x / (1.0 + jnp.exp(-x))


def _prep_body(f_ref, xyz_ref, wc_ref, wd_ref, wdg_ref, wnrg_ref, wrow_ref,
               a_ref, g_ref, r_ref):
    f = f_ref[...]
    xyz = xyz_ref[...]
    r = jnp.sqrt(jnp.sum(xyz * xyz, axis=1, keepdims=True))
    a_ref[...] = (jnp.dot(f, wc_ref[...] - wd_ref[...],
                          preferred_element_type=jnp.float32)
                  + r * wrow_ref[0:1, :])
    g_ref[...] = (jnp.dot(f, wdg_ref[...], preferred_element_type=jnp.float32)
                  + r * wnrg_ref[...])
    r_ref[...] = r


def _post_body(p_ref, f_ref, w2_ref, o_ref):
    t = jnp.dot(p_ref[...], w2_ref[...], preferred_element_type=jnp.float32) + f_ref[...]
    o_ref[...] = t * jax.lax.logistic(t)


CHUNK = 8
NB = 2
FAST_CORE = 1
NPW_F = 400
NPW_S = 240


def _make_edge_kernel(n_pad, k_nb, c_dim):
    nv = c_dim // L
    ng = k_nb // L
    span = NPW_F + NPW_S
    mesh = plsc.VectorSubcoreMesh(core_axis_name="c", subcore_axis_name="s")

    @functools.partial(
        pl.kernel,
        out_type=jax.ShapeDtypeStruct((n_pad, c_dim), jnp.float32),
        mesh=mesh,
        scratch_types=[
            pltpu.VMEM((NPW_F * k_nb,), jnp.int32),
            pltpu.VMEM((2, CHUNK, c_dim), jnp.float32),
            pltpu.VMEM((n_pad,), jnp.float32),
            pltpu.VMEM((n_pad,), jnp.float32),
            pltpu.VMEM((n_pad,), jnp.float32),
            pltpu.VMEM((n_pad,), jnp.float32),
            pltpu.VMEM((c_dim,), jnp.float32),
            pltpu.VMEM((c_dim,), jnp.float32),
            pltpu.VMEM((c_dim,), jnp.float32),
            pltpu.VMEM((2, NB * k_nb, c_dim), jnp.float32),
            pltpu.VMEM((CHUNK, c_dim), jnp.float32),
            pltpu.VMEM((3, k_nb + L), jnp.float32),
            pltpu.SemaphoreType.DMA,
            pltpu.SemaphoreType.DMA,
            pltpu.SemaphoreType.DMA,
            pltpu.SemaphoreType.DMA,
        ],
        compiler_params=pltpu.CompilerParams(needs_layout_passes=False),
    )
    def edge_kernel(g_hbm, a_hbm, idx_hbm, sx_hbm, sy_hbm, sz_hbm, sr_hbm,
                    wrr_hbm, wdot_hbm, wcos_hbm, p_hbm,
                    idx_v, abuf, sx_v, sy_v, sz_v, sr_v, wrr_v, wdot_v, wcos_v,
                    gbuf, pchunk, coef, sem0, sem1, asem0, asem1):
        cid = lax.axis_index("c")
        sid = lax.axis_index("s")
        fast = cid == FAST_CORE
        base = sid * span + jnp.where(fast, 0, NPW_F)
        nch = jnp.where(fast, NPW_F // CHUNK, NPW_S // CHUNK)
        base_lim = jnp.where(fast, NPW_F - NB, NPW_S - NB)
        pltpu.sync_copy(idx_hbm.at[pl.ds(base * k_nb, NPW_F * k_nb)], idx_v)
        pltpu.sync_copy(sx_hbm, sx_v)
        pltpu.sync_copy(sy_hbm, sy_v)
        pltpu.sync_copy(sz_hbm, sz_v)
        pltpu.sync_copy(sr_hbm, sr_v)
        pltpu.sync_copy(wrr_hbm, wrr_v)
        pltpu.sync_copy(wdot_hbm, wdot_v)
        pltpu.sync_copy(wcos_hbm, wcos_v)

        wrr = [wrr_v[pl.ds(v * L, L)] for v in range(nv)]
        wdot = [wdot_v[pl.ds(v * L, L)] for v in range(nv)]
        wcos = [wcos_v[pl.ds(v * L, L)] for v in range(nv)]
        inv_k = jnp.float32(1.0 / k_nb)
        sems = (sem0, sem1)
        asems = (asem0, asem1)

        pltpu.async_copy(g_hbm.at[idx_v.at[pl.ds(0, NB * k_nb)]],
                         gbuf.at[0], sems[0])
        pltpu.async_copy(a_hbm.at[pl.ds(base, CHUNK)], abuf.at[0], asems[0])

        @pl.loop(0, nch // 2)
        def chunk_loop(c2):
            for cc in range(2):
                c = c2 * 2 + cc
                i0 = c * CHUNK
                c1 = jnp.minimum(c + 1, nch - 1)
                pltpu.async_copy(a_hbm.at[pl.ds(base + c1 * CHUNK, CHUNK)],
                                 abuf.at[(cc + 1) % 2], asems[(cc + 1) % 2])
                pltpu.make_async_copy(a_hbm.at[pl.ds(0, CHUNK)],
                                      abuf.at[cc], asems[cc]).wait()
                for jb in range(CHUNK // NB):
                    ib = i0 + jb * NB
                    p = jb % 2
                    pn = (jb + 1) % 2
                    ibn = jnp.minimum(ib + NB, base_lim)
                    pltpu.async_copy(
                        g_hbm.at[idx_v.at[pl.ds(ibn * k_nb, NB * k_nb)]],
                        gbuf.at[pn], sems[pn])
                    pltpu.make_async_copy(g_hbm.at[idx_v.at[pl.ds(0, NB * k_nb)]],
                                          gbuf.at[p], sems[p]).wait()
                    for jn in range(NB):
                        i = ib + jn
                        j = jb * NB + jn
                        koff = jn * k_nb
                        g = base + i
                        idxg = jnp.full((L,), g, jnp.int32)
                        xn = plsc.load_gather(sx_v, [idxg])
                        yn = plsc.load_gather(sy_v, [idxg])
                        zn = plsc.load_gather(sz_v, [idxg])
                        rn = plsc.load_gather(sr_v, [idxg])
                        for grp in range(ng):
                            idxv = idx_v[pl.ds(i * k_nb + grp * L, L)]
                            xj = plsc.load_gather(sx_v, [idxv])
                            yj = plsc.load_gather(sy_v, [idxv])
                            zj = plsc.load_gather(sz_v, [idxv])
                            rj = plsc.load_gather(sr_v, [idxv])
                            dot = xn * xj + yn * yj + zn * zj
                            dx = xj - xn
                            dy = yj - yn
                            dz = zj - zn
                            coef[0, pl.ds(grp * L, L)] = _sqrt16(
                                dx * dx + dy * dy + dz * dz)
                            coef[1, pl.ds(grp * L, L)] = dot
                            coef[2, pl.ds(grp * L, L)] = dot / (rn * rj + EPS)
                        av = [abuf[cc, j, pl.ds(v * L, L)] for v in range(nv)]

                        def edge_body(k, acc, _p=p, _koff=koff):
                            crr = coef[0, pl.ds(k, L)][0]
                            cdot = coef[1, pl.ds(k, L)][0]
                            ccos = coef[2, pl.ds(k, L)][0]
                            out = []
                            for v in range(nv):
                                gv = gbuf[_p, _koff + k, pl.ds(v * L, L)]
                                pre = (av[v] + gv + crr * wrr[v]
                                       + cdot * wdot[v] + ccos * wcos[v])
                                out.append(acc[v] + _silu16(pre))
                            return tuple(out)

                        acc = lax.fori_loop(
                            0, k_nb, edge_body,
                            tuple(jnp.zeros((L,), jnp.float32)
                                  for _ in range(nv)),
                            unroll=2)
                        for v in range(nv):
                            pchunk[j, pl.ds(v * L, L)] = acc[v] * inv_k
                pltpu.sync_copy(pchunk, p_hbm.at[pl.ds(base + i0, CHUNK)])

        pltpu.make_async_copy(g_hbm.at[idx_v.at[pl.ds(0, NB * k_nb)]],
                              gbuf.at[0], sems[0]).wait()
        pltpu.make_async_copy(a_hbm.at[pl.ds(0, CHUNK)],
                              abuf.at[0], asems[0]).wait()

    return edge_kernel


def kernel(feat, xyz_centered, idx_knn, W1, W2):
    B, N, C = feat.shape
    K = idx_knn.shape[-1]
    assert B == 1 and C % L == 0 and K % L == 0
    npw = -(-N // NW)
    npw = -(-npw // 8) * 8
    n_pad = npw * NW
    pad = n_pad - N

    f_p = jnp.pad(feat[0], ((0, pad), (0, 0)))
    xyz_p = jnp.pad(xyz_centered[0], ((0, pad), (0, 0)))
    idx_p = jnp.pad(idx_knn[0], ((0, pad), (0, 0)))
    wc = W1[:C]
    wd = W1[C:2 * C]
    wrow = W1[2 * C:]
    src = np.empty((C,), np.int32)
    t16 = np.arange(L)
    for blk in range(C // (2 * L)):
        src[blk * 2 * L + 2 * t16] = blk * 2 * L + t16
        src[blk * 2 * L + 2 * t16 + 1] = blk * 2 * L + L + t16

    a_t, g_t, r_t = pl.pallas_call(
        _prep_body,
        out_shape=[
            jax.ShapeDtypeStruct((n_pad, C), jnp.float32),
            jax.ShapeDtypeStruct((n_pad, C), jnp.float32),
            jax.ShapeDtypeStruct((n_pad, 1), jnp.float32),
        ],
    )(f_p, xyz_p, wc, wd, wd, wrow[1][None, :], wrow)

    assert n_pad == NS * (NPW_F + NPW_S)
    idx_rows = (NS - 1) * (NPW_F + NPW_S) + 2 * NPW_F
    idx_flat = jnp.pad(idx_p.reshape(-1), (0, (idx_rows - n_pad) * K))
    edge = _make_edge_kernel(n_pad, K, C)
    p_t = edge(g_t, a_t, idx_flat,
               xyz_p[:, 0], xyz_p[:, 1], xyz_p[:, 2],
               r_t.reshape(n_pad),
               wrow[2], wrow[3], wrow[4])

    out = pl.pallas_call(
        _post_body,
        out_shape=jax.ShapeDtypeStruct((n_pad, C), jnp.float32),
    )(p_t, f_p, W2)
    return out[:N][None]


# speedup vs baseline: 1.0380x; 1.0380x over previous
import functools

import jax
import jax.numpy as jnp
import numpy as np
from jax import lax
from jax.experimental import pallas as pl
from jax.experimental.pallas import tpu as pltpu
from jax.experimental.pallas import tpu_sc as plsc

EPS = 1e-06
L = 16
NC = 2
NS = 16
NW = NC * NS


def _sqrt16(x):
    i = plsc.bitcast(x, jnp.int32)
    y = plsc.bitcast((i >> 1) + jnp.int32(0x1FBD1DF5), jnp.float32)
    y = 0.5 * (y + x / y)
    y = 0.5 * (y + x / y)
    return y


def _silu16(x):
    return x / (1.0 + jnp.exp(-x))


def _prep_body(f_ref, xyz_ref, wc_ref, wd_ref, wdg_ref, wnrg_ref, wrow_ref,
               a_ref, g_ref, r_ref):
    f = f_ref[...]
    xyz = xyz_ref[...]
    r = jnp.sqrt(jnp.sum(xyz * xyz, axis=1, keepdims=True))
    a_ref[...] = (jnp.dot(f, wc_ref[...] - wd_ref[...],
                          preferred_element_type=jnp.float32)
                  + r * wrow_ref[0:1, :])
    g_ref[...] = (jnp.dot(f, wdg_ref[...], preferred_element_type=jnp.float32)
                  + r * wnrg_ref[...])
    r_ref[...] = r


def _post_body(p_ref, f_ref, w2_ref, o_ref):
    t = jnp.dot(p_ref[...], w2_ref[...], preferred_element_type=jnp.float32) + f_ref[...]
    o_ref[...] = t * jax.lax.logistic(t)


CHUNK = 8
NB = 2
FAST_CORE = 1
NPW_F = 432
NPW_S = 208


def _make_edge_kernel(n_pad, k_nb, c_dim):
    nv = c_dim // L
    ng = k_nb // L
    span = NPW_F + NPW_S
    mesh = plsc.VectorSubcoreMesh(core_axis_name="c", subcore_axis_name="s")

    @functools.partial(
        pl.kernel,
        out_type=jax.ShapeDtypeStruct((n_pad, c_dim), jnp.float32),
        mesh=mesh,
        scratch_types=[
            pltpu.VMEM((NPW_F * k_nb,), jnp.int32),
            pltpu.VMEM((2, CHUNK, c_dim), jnp.float32),
            pltpu.VMEM((n_pad,), jnp.float32),
            pltpu.VMEM((n_pad,), jnp.float32),
            pltpu.VMEM((n_pad,), jnp.float32),
            pltpu.VMEM((n_pad,), jnp.float32),
            pltpu.VMEM((c_dim,), jnp.float32),
            pltpu.VMEM((c_dim,), jnp.float32),
            pltpu.VMEM((c_dim,), jnp.float32),
            pltpu.VMEM((2, NB * k_nb, c_dim), jnp.float32),
            pltpu.VMEM((CHUNK, c_dim), jnp.float32),
            pltpu.VMEM((3, k_nb + L), jnp.float32),
            pltpu.SemaphoreType.DMA,
            pltpu.SemaphoreType.DMA,
            pltpu.SemaphoreType.DMA,
            pltpu.SemaphoreType.DMA,
        ],
        compiler_params=pltpu.CompilerParams(needs_layout_passes=False),
    )
    def edge_kernel(g_hbm, a_hbm, idx_hbm, sx_hbm, sy_hbm, sz_hbm, sr_hbm,
                    wrr_hbm, wdot_hbm, wcos_hbm, p_hbm,
                    idx_v, abuf, sx_v, sy_v, sz_v, sr_v, wrr_v, wdot_v, wcos_v,
                    gbuf, pchunk, coef, sem0, sem1, asem0, asem1):
        cid = lax.axis_index("c")
        sid = lax.axis_index("s")
        fast = cid == FAST_CORE
        base = sid * span + jnp.where(fast, 0, NPW_F)
        nch = jnp.where(fast, NPW_F // CHUNK, NPW_S // CHUNK)
        base_lim = jnp.where(fast, NPW_F - NB, NPW_S - NB)
        pltpu.sync_copy(idx_hbm.at[pl.ds(base * k_nb, NPW_F * k_nb)], idx_v)
        pltpu.sync_copy(sx_hbm, sx_v)
        pltpu.sync_copy(sy_hbm, sy_v)
        pltpu.sync_copy(sz_hbm, sz_v)
        pltpu.sync_copy(sr_hbm, sr_v)
        pltpu.sync_copy(wrr_hbm, wrr_v)
        pltpu.sync_copy(wdot_hbm, wdot_v)
        pltpu.sync_copy(wcos_hbm, wcos_v)

        wrr = [wrr_v[pl.ds(v * L, L)] for v in range(nv)]
        wdot = [wdot_v[pl.ds(v * L, L)] for v in range(nv)]
        wcos = [wcos_v[pl.ds(v * L, L)] for v in range(nv)]
        inv_k = jnp.float32(1.0 / k_nb)
        sems = (sem0, sem1)
        asems = (asem0, asem1)

        pltpu.async_copy(g_hbm.at[idx_v.at[pl.ds(0, NB * k_nb)]],
                         gbuf.at[0], sems[0])
        pltpu.async_copy(a_hbm.at[pl.ds(base, CHUNK)], abuf.at[0], asems[0])

        @pl.loop(0, nch // 2)
        def chunk_loop(c2):
            for cc in range(2):
                c = c2 * 2 + cc
                i0 = c * CHUNK
                c1 = jnp.minimum(c + 1, nch - 1)
                pltpu.async_copy(a_hbm.at[pl.ds(base + c1 * CHUNK, CHUNK)],
                                 abuf.at[(cc + 1) % 2], asems[(cc + 1) % 2])
                pltpu.make_async_copy(a_hbm.at[pl.ds(0, CHUNK)],
                                      abuf.at[cc], asems[cc]).wait()
                for jb in range(CHUNK // NB):
                    ib = i0 + jb * NB
                    p = jb % 2
                    pn = (jb + 1) % 2
                    ibn = jnp.minimum(ib + NB, base_lim)
                    pltpu.async_copy(
                        g_hbm.at[idx_v.at[pl.ds(ibn * k_nb, NB * k_nb)]],
                        gbuf.at[pn], sems[pn])
                    pltpu.make_async_copy(g_hbm.at[idx_v.at[pl.ds(0, NB * k_nb)]],
                                          gbuf.at[p], sems[p]).wait()
                    for jn in range(NB):
                        i = ib + jn
                        j = jb * NB + jn
                        koff = jn * k_nb
                        g = base + i
                        idxg = jnp.full((L,), g, jnp.int32)
                        xn = plsc.load_gather(sx_v, [idxg])
                        yn = plsc.load_gather(sy_v, [idxg])
                        zn = plsc.load_gather(sz_v, [idxg])
                        rn = plsc.load_gather(sr_v, [idxg])
                        for grp in range(ng):
                            idxv = idx_v[pl.ds(i * k_nb + grp * L, L)]
                            xj = plsc.load_gather(sx_v, [idxv])
                            yj = plsc.load_gather(sy_v, [idxv])
                            zj = plsc.load_gather(sz_v, [idxv])
                            rj = plsc.load_gather(sr_v, [idxv])
                            dot = xn * xj + yn * yj + zn * zj
                            dx = xj - xn
                            dy = yj - yn
                            dz = zj - zn
                            coef[0, pl.ds(grp * L, L)] = _sqrt16(
                                dx * dx + dy * dy + dz * dz)
                            coef[1, pl.ds(grp * L, L)] = dot
                            coef[2, pl.ds(grp * L, L)] = dot / (rn * rj + EPS)
                        av = [abuf[cc, j, pl.ds(v * L, L)] for v in range(nv)]

                        def edge_body(k, acc, _p=p, _koff=koff):
                            crr = coef[0, pl.ds(k, L)][0]
                            cdot = coef[1, pl.ds(k, L)][0]
                            ccos = coef[2, pl.ds(k, L)][0]
                            out = []
                            for v in range(nv):
                                gv = gbuf[_p, _koff + k, pl.ds(v * L, L)]
                                pre = (av[v] + gv + crr * wrr[v]
                                       + cdot * wdot[v] + ccos * wcos[v])
                                out.append(acc[v] + _silu16(pre))
                            return tuple(out)

                        acc = lax.fori_loop(
                            0, k_nb, edge_body,
                            tuple(jnp.zeros((L,), jnp.float32)
                                  for _ in range(nv)),
                            unroll=2)
                        for v in range(nv):
                            pchunk[j, pl.ds(v * L, L)] = acc[v] * inv_k
                pltpu.sync_copy(pchunk, p_hbm.at[pl.ds(base + i0, CHUNK)])

        pltpu.make_async_copy(g_hbm.at[idx_v.at[pl.ds(0, NB * k_nb)]],
                              gbuf.at[0], sems[0]).wait()
        pltpu.make_async_copy(a_hbm.at[pl.ds(0, CHUNK)],
                              abuf.at[0], asems[0]).wait()

    return edge_kernel


def kernel(feat, xyz_centered, idx_knn, W1, W2):
    B, N, C = feat.shape
    K = idx_knn.shape[-1]
    assert B == 1 and C % L == 0 and K % L == 0
    npw = -(-N // NW)
    npw = -(-npw // 8) * 8
    n_pad = npw * NW
    pad = n_pad - N

    f_p = jnp.pad(feat[0], ((0, pad), (0, 0)))
    xyz_p = jnp.pad(xyz_centered[0], ((0, pad), (0, 0)))
    idx_p = jnp.pad(idx_knn[0], ((0, pad), (0, 0)))
    wc = W1[:C]
    wd = W1[C:2 * C]
    wrow = W1[2 * C:]
    src = np.empty((C,), np.int32)
    t16 = np.arange(L)
    for blk in range(C // (2 * L)):
        src[blk * 2 * L + 2 * t16] = blk * 2 * L + t16
        src[blk * 2 * L + 2 * t16 + 1] = blk * 2 * L + L + t16

    a_t, g_t, r_t = pl.pallas_call(
        _prep_body,
        out_shape=[
            jax.ShapeDtypeStruct((n_pad, C), jnp.float32),
            jax.ShapeDtypeStruct((n_pad, C), jnp.float32),
            jax.ShapeDtypeStruct((n_pad, 1), jnp.float32),
        ],
    )(f_p, xyz_p, wc, wd, wd, wrow[1][None, :], wrow)

    assert n_pad == NS * (NPW_F + NPW_S)
    idx_rows = (NS - 1) * (NPW_F + NPW_S) + 2 * NPW_F
    idx_flat = jnp.pad(idx_p.reshape(-1), (0, (idx_rows - n_pad) * K))
    edge = _make_edge_kernel(n_pad, K, C)
    p_t = edge(g_t, a_t, idx_flat,
               xyz_p[:, 0], xyz_p[:, 1], xyz_p[:, 2],
               r_t.reshape(n_pad),
               wrow[2], wrow[3], wrow[4])

    out = pl.pallas_call(
        _post_body,
        out_shape=jax.ShapeDtypeStruct((n_pad, C), jnp.float32),
    )(p_t, f_p, W2)
    return out[:N][None]


# speedup vs baseline: 1.0444x; 1.0062x over previous
import functools

import jax
import jax.numpy as jnp
import numpy as np
from jax import lax
from jax.experimental import pallas as pl
from jax.experimental.pallas import tpu as pltpu
from jax.experimental.pallas import tpu_sc as plsc

EPS = 1e-06
L = 16
NC = 2
NS = 16
NW = NC * NS


def _sqrt16(x):
    i = plsc.bitcast(x, jnp.int32)
    y = plsc.bitcast((i >> 1) + jnp.int32(0x1FBD1DF5), jnp.float32)
    y = 0.5 * (y + x / y)
    y = 0.5 * (y + x / y)
    return y


def _silu16(x):
    return x / (1.0 + jnp.exp(-x))


def _prep_body(f_ref, xyz_ref, wc_ref, wd_ref, wdg_ref, wnrg_ref, wrow_ref,
               a_ref, g_ref, r_ref):
    f = f_ref[...]
    xyz = xyz_ref[...]
    r = jnp.sqrt(jnp.sum(xyz * xyz, axis=1, keepdims=True))
    a_ref[...] = (jnp.dot(f, wc_ref[...] - wd_ref[...],
                          preferred_element_type=jnp.float32)
                  + r * wrow_ref[0:1, :])
    g_ref[...] = (jnp.dot(f, wdg_ref[...], preferred_element_type=jnp.float32)
                  + r * wnrg_ref[...])
    r_ref[...] = r


def _post_body(p_ref, f_ref, w2_ref, o_ref):
    t = jnp.dot(p_ref[...], w2_ref[...], preferred_element_type=jnp.float32) + f_ref[...]
    o_ref[...] = t * jax.lax.logistic(t)


CHUNK = 8
NB = 2
FAST_CORE = 1
NPW_F = 416
NPW_S = 224


def _make_edge_kernel(n_pad, k_nb, c_dim):
    nv = c_dim // L
    ng = k_nb // L
    span = NPW_F + NPW_S
    mesh = plsc.VectorSubcoreMesh(core_axis_name="c", subcore_axis_name="s")

    @functools.partial(
        pl.kernel,
        out_type=jax.ShapeDtypeStruct((n_pad, c_dim), jnp.float32),
        mesh=mesh,
        scratch_types=[
            pltpu.VMEM((NPW_F * k_nb,), jnp.int32),
            pltpu.VMEM((2, CHUNK, c_dim), jnp.float32),
            pltpu.VMEM((n_pad,), jnp.float32),
            pltpu.VMEM((n_pad,), jnp.float32),
            pltpu.VMEM((n_pad,), jnp.float32),
            pltpu.VMEM((n_pad,), jnp.float32),
            pltpu.VMEM((c_dim,), jnp.float32),
            pltpu.VMEM((c_dim,), jnp.float32),
            pltpu.VMEM((c_dim,), jnp.float32),
            pltpu.VMEM((2, NB * k_nb, c_dim), jnp.float32),
            pltpu.VMEM((CHUNK, c_dim), jnp.float32),
            pltpu.VMEM((3, k_nb + L), jnp.float32),
            pltpu.SemaphoreType.DMA,
            pltpu.SemaphoreType.DMA,
            pltpu.SemaphoreType.DMA,
            pltpu.SemaphoreType.DMA,
        ],
        compiler_params=pltpu.CompilerParams(needs_layout_passes=False),
    )
    def edge_kernel(g_hbm, a_hbm, idx_hbm, sx_hbm, sy_hbm, sz_hbm, sr_hbm,
                    wrr_hbm, wdot_hbm, wcos_hbm, p_hbm,
                    idx_v, abuf, sx_v, sy_v, sz_v, sr_v, wrr_v, wdot_v, wcos_v,
                    gbuf, pchunk, coef, sem0, sem1, asem0, asem1):
        cid = lax.axis_index("c")
        sid = lax.axis_index("s")
        fast = cid == FAST_CORE
        base = sid * span + jnp.where(fast, 0, NPW_F)
        nch = jnp.where(fast, NPW_F // CHUNK, NPW_S // CHUNK)
        base_lim = jnp.where(fast, NPW_F - NB, NPW_S - NB)
        pltpu.sync_copy(idx_hbm.at[pl.ds(base * k_nb, NPW_F * k_nb)], idx_v)
        pltpu.sync_copy(sx_hbm, sx_v)
        pltpu.sync_copy(sy_hbm, sy_v)
        pltpu.sync_copy(sz_hbm, sz_v)
        pltpu.sync_copy(sr_hbm, sr_v)
        pltpu.sync_copy(wrr_hbm, wrr_v)
        pltpu.sync_copy(wdot_hbm, wdot_v)
        pltpu.sync_copy(wcos_hbm, wcos_v)

        wrr = [wrr_v[pl.ds(v * L, L)] for v in range(nv)]
        wdot = [wdot_v[pl.ds(v * L, L)] for v in range(nv)]
        wcos = [wcos_v[pl.ds(v * L, L)] for v in range(nv)]
        inv_k = jnp.float32(1.0 / k_nb)
        sems = (sem0, sem1)
        asems = (asem0, asem1)

        pltpu.async_copy(g_hbm.at[idx_v.at[pl.ds(0, NB * k_nb)]],
                         gbuf.at[0], sems[0])
        pltpu.async_copy(a_hbm.at[pl.ds(base, CHUNK)], abuf.at[0], asems[0])

        @pl.loop(0, nch // 2)
        def chunk_loop(c2):
            for cc in range(2):
                c = c2 * 2 + cc
                i0 = c * CHUNK
                c1 = jnp.minimum(c + 1, nch - 1)
                pltpu.async_copy(a_hbm.at[pl.ds(base + c1 * CHUNK, CHUNK)],
                                 abuf.at[(cc + 1) % 2], asems[(cc + 1) % 2])
                pltpu.make_async_copy(a_hbm.at[pl.ds(0, CHUNK)],
                                      abuf.at[cc], asems[cc]).wait()
                for jb in range(CHUNK // NB):
                    ib = i0 + jb * NB
                    p = jb % 2
                    pn = (jb + 1) % 2
                    ibn = jnp.minimum(ib + NB, base_lim)
                    pltpu.async_copy(
                        g_hbm.at[idx_v.at[pl.ds(ibn * k_nb, NB * k_nb)]],
                        gbuf.at[pn], sems[pn])
                    pltpu.make_async_copy(g_hbm.at[idx_v.at[pl.ds(0, NB * k_nb)]],
                                          gbuf.at[p], sems[p]).wait()
                    for jn in range(NB):
                        i = ib + jn
                        j = jb * NB + jn
                        koff = jn * k_nb
                        g = base + i
                        idxg = jnp.full((L,), g, jnp.int32)
                        xn = plsc.load_gather(sx_v, [idxg])
                        yn = plsc.load_gather(sy_v, [idxg])
                        zn = plsc.load_gather(sz_v, [idxg])
                        rn = plsc.load_gather(sr_v, [idxg])
                        for grp in range(ng):
                            idxv = idx_v[pl.ds(i * k_nb + grp * L, L)]
                            xj = plsc.load_gather(sx_v, [idxv])
                            yj = plsc.load_gather(sy_v, [idxv])
                            zj = plsc.load_gather(sz_v, [idxv])
                            rj = plsc.load_gather(sr_v, [idxv])
                            dot = xn * xj + yn * yj + zn * zj
                            dx = xj - xn
                            dy = yj - yn
                            dz = zj - zn
                            coef[0, pl.ds(grp * L, L)] = _sqrt16(
                                dx * dx + dy * dy + dz * dz)
                            coef[1, pl.ds(grp * L, L)] = dot
                            coef[2, pl.ds(grp * L, L)] = dot / (rn * rj + EPS)
                        av = [abuf[cc, j, pl.ds(v * L, L)] for v in range(nv)]

                        def edge_body(k, acc, _p=p, _koff=koff):
                            crr = coef[0, pl.ds(k, L)][0]
                            cdot = coef[1, pl.ds(k, L)][0]
                            ccos = coef[2, pl.ds(k, L)][0]
                            out = []
                            for v in range(nv):
                                gv = gbuf[_p, _koff + k, pl.ds(v * L, L)]
                                pre = (av[v] + gv + crr * wrr[v]
                                       + cdot * wdot[v] + ccos * wcos[v])
                                out.append(acc[v] + _silu16(pre))
                            return tuple(out)

                        acc = lax.fori_loop(
                            0, k_nb, edge_body,
                            tuple(jnp.zeros((L,), jnp.float32)
                                  for _ in range(nv)),
                            unroll=2)
                        for v in range(nv):
                            pchunk[j, pl.ds(v * L, L)] = acc[v] * inv_k
                pltpu.sync_copy(pchunk, p_hbm.at[pl.ds(base + i0, CHUNK)])

        pltpu.make_async_copy(g_hbm.at[idx_v.at[pl.ds(0, NB * k_nb)]],
                              gbuf.at[0], sems[0]).wait()
        pltpu.make_async_copy(a_hbm.at[pl.ds(0, CHUNK)],
                              abuf.at[0], asems[0]).wait()

    return edge_kernel


def kernel(feat, xyz_centered, idx_knn, W1, W2):
    B, N, C = feat.shape
    K = idx_knn.shape[-1]
    assert B == 1 and C % L == 0 and K % L == 0
    npw = -(-N // NW)
    npw = -(-npw // 8) * 8
    n_pad = npw * NW
    pad = n_pad - N

    f_p = jnp.pad(feat[0], ((0, pad), (0, 0)))
    xyz_p = jnp.pad(xyz_centered[0], ((0, pad), (0, 0)))
    idx_p = jnp.pad(idx_knn[0], ((0, pad), (0, 0)))
    wc = W1[:C]
    wd = W1[C:2 * C]
    wrow = W1[2 * C:]
    src = np.empty((C,), np.int32)
    t16 = np.arange(L)
    for blk in range(C // (2 * L)):
        src[blk * 2 * L + 2 * t16] = blk * 2 * L + t16
        src[blk * 2 * L + 2 * t16 + 1] = blk * 2 * L + L + t16

    a_t, g_t, r_t = pl.pallas_call(
        _prep_body,
        out_shape=[
            jax.ShapeDtypeStruct((n_pad, C), jnp.float32),
            jax.ShapeDtypeStruct((n_pad, C), jnp.float32),
            jax.ShapeDtypeStruct((n_pad, 1), jnp.float32),
        ],
    )(f_p, xyz_p, wc, wd, wd, wrow[1][None, :], wrow)

    assert n_pad == NS * (NPW_F + NPW_S)
    idx_rows = (NS - 1) * (NPW_F + NPW_S) + 2 * NPW_F
    idx_flat = jnp.pad(idx_p.reshape(-1), (0, (idx_rows - n_pad) * K))
    edge = _make_edge_kernel(n_pad, K, C)
    p_t = edge(g_t, a_t, idx_flat,
               xyz_p[:, 0], xyz_p[:, 1], xyz_p[:, 2],
               r_t.reshape(n_pad),
               wrow[2], wrow[3], wrow[4])

    out = pl.pallas_call(
        _post_body,
        out_shape=jax.ShapeDtypeStruct((n_pad, C), jnp.float32),
    )(p_t, f_p, W2)
    return out[:N][None]


# speedup vs baseline: 1.0968x; 1.0502x over previous
import functools

import jax
import jax.numpy as jnp
from jax import lax
from jax.experimental import pallas as pl
from jax.experimental.pallas import tpu as pltpu
from jax.experimental.pallas import tpu_sc as plsc

EPS = 1e-06
L = 16
NC = 2
NS = 16
NW = NC * NS


def _sqrt16(x):
    i = plsc.bitcast(x, jnp.int32)
    y = plsc.bitcast((i >> 1) + jnp.int32(0x1FBD1DF5), jnp.float32)
    y = 0.5 * (y + x / y)
    y = 0.5 * (y + x / y)
    return y


def _silu16(x):
    return x / (1.0 + jnp.exp(-x))


def _prep_body(f_ref, xyz_ref, wc_ref, wd_ref, wrow_ref, a_ref, g_ref, r_ref):
    f = f_ref[...]
    xyz = xyz_ref[...]
    r = jnp.sqrt(jnp.sum(xyz * xyz, axis=1, keepdims=True))
    wd = wd_ref[...]
    a_ref[...] = (jnp.dot(f, wc_ref[...] - wd, preferred_element_type=jnp.float32)
                  + r * wrow_ref[0:1, :])
    g_ref[...] = (jnp.dot(f, wd, preferred_element_type=jnp.float32)
                  + r * wrow_ref[1:2, :])
    r_ref[...] = r


def _post_body(p_ref, f_ref, w2_ref, o_ref):
    t = jnp.dot(p_ref[...], w2_ref[...], preferred_element_type=jnp.float32) + f_ref[...]
    o_ref[...] = t * jax.lax.logistic(t)


CHUNK = 8
NB = 2
FAST_CORE = 1
NPW_F = 416
NPW_S = 224


def _make_edge_kernel(n_pad, k_nb, c_dim):
    nv = c_dim // L
    ng = k_nb // L
    span = NPW_F + NPW_S
    mesh = plsc.VectorSubcoreMesh(core_axis_name="c", subcore_axis_name="s")

    @functools.partial(
        pl.kernel,
        out_type=jax.ShapeDtypeStruct((n_pad, c_dim), jnp.float32),
        mesh=mesh,
        scratch_types=[
            pltpu.VMEM((NPW_F * k_nb,), jnp.int32),
            pltpu.VMEM((2, CHUNK, c_dim), jnp.float32),
            pltpu.VMEM((n_pad,), jnp.float32),
            pltpu.VMEM((n_pad,), jnp.float32),
            pltpu.VMEM((n_pad,), jnp.float32),
            pltpu.VMEM((n_pad,), jnp.float32),
            pltpu.VMEM((c_dim,), jnp.float32),
            pltpu.VMEM((c_dim,), jnp.float32),
            pltpu.VMEM((c_dim,), jnp.float32),
            pltpu.VMEM((2, NB * k_nb, c_dim), jnp.float32),
            pltpu.VMEM((CHUNK, c_dim), jnp.float32),
            pltpu.VMEM((3, k_nb + L), jnp.float32),
            pltpu.SemaphoreType.DMA,
            pltpu.SemaphoreType.DMA,
            pltpu.SemaphoreType.DMA,
            pltpu.SemaphoreType.DMA,
        ],
        compiler_params=pltpu.CompilerParams(needs_layout_passes=False),
    )
    def edge_kernel(g_hbm, a_hbm, idx_hbm, sx_hbm, sy_hbm, sz_hbm, sr_hbm,
                    wrr_hbm, wdot_hbm, wcos_hbm, p_hbm,
                    idx_v, abuf, sx_v, sy_v, sz_v, sr_v, wrr_v, wdot_v, wcos_v,
                    gbuf, pchunk, coef, sem0, sem1, asem0, asem1):
        cid = lax.axis_index("c")
        sid = lax.axis_index("s")
        fast = cid == FAST_CORE
        base = sid * span + jnp.where(fast, 0, NPW_F)
        nch = jnp.where(fast, NPW_F // CHUNK, NPW_S // CHUNK)
        base_lim = jnp.where(fast, NPW_F - NB, NPW_S - NB)
        pltpu.sync_copy(idx_hbm.at[pl.ds(base * k_nb, NPW_F * k_nb)], idx_v)
        pltpu.sync_copy(sx_hbm, sx_v)
        pltpu.sync_copy(sy_hbm, sy_v)
        pltpu.sync_copy(sz_hbm, sz_v)
        pltpu.sync_copy(sr_hbm, sr_v)
        pltpu.sync_copy(wrr_hbm, wrr_v)
        pltpu.sync_copy(wdot_hbm, wdot_v)
        pltpu.sync_copy(wcos_hbm, wcos_v)

        wrr = [wrr_v[pl.ds(v * L, L)] for v in range(nv)]
        wdot = [wdot_v[pl.ds(v * L, L)] for v in range(nv)]
        wcos = [wcos_v[pl.ds(v * L, L)] for v in range(nv)]
        inv_k = jnp.float32(1.0 / k_nb)
        sems = (sem0, sem1)
        asems = (asem0, asem1)

        pltpu.async_copy(g_hbm.at[idx_v.at[pl.ds(0, NB * k_nb)]],
                         gbuf.at[0], sems[0])
        pltpu.async_copy(a_hbm.at[pl.ds(base, CHUNK)], abuf.at[0], asems[0])

        @pl.loop(0, nch // 2)
        def chunk_loop(c2):
            for cc in range(2):
                c = c2 * 2 + cc
                i0 = c * CHUNK
                c1 = jnp.minimum(c + 1, nch - 1)
                pltpu.async_copy(a_hbm.at[pl.ds(base + c1 * CHUNK, CHUNK)],
                                 abuf.at[(cc + 1) % 2], asems[(cc + 1) % 2])
                pltpu.make_async_copy(a_hbm.at[pl.ds(0, CHUNK)],
                                      abuf.at[cc], asems[cc]).wait()
                for jb in range(CHUNK // NB):
                    ib = i0 + jb * NB
                    p = jb % 2
                    pn = (jb + 1) % 2
                    ibn = jnp.minimum(ib + NB, base_lim)
                    pltpu.async_copy(
                        g_hbm.at[idx_v.at[pl.ds(ibn * k_nb, NB * k_nb)]],
                        gbuf.at[pn], sems[pn])
                    pltpu.make_async_copy(g_hbm.at[idx_v.at[pl.ds(0, NB * k_nb)]],
                                          gbuf.at[p], sems[p]).wait()
                    for jn in range(NB):
                        i = ib + jn
                        j = jb * NB + jn
                        koff = jn * k_nb
                        g = base + i
                        idxg = jnp.full((L,), g, jnp.int32)
                        xn = plsc.load_gather(sx_v, [idxg])
                        yn = plsc.load_gather(sy_v, [idxg])
                        zn = plsc.load_gather(sz_v, [idxg])
                        rn = plsc.load_gather(sr_v, [idxg])
                        for grp in range(ng):
                            idxv = idx_v[pl.ds(i * k_nb + grp * L, L)]
                            xj = plsc.load_gather(sx_v, [idxv])
                            yj = plsc.load_gather(sy_v, [idxv])
                            zj = plsc.load_gather(sz_v, [idxv])
                            rj = plsc.load_gather(sr_v, [idxv])
                            dot = xn * xj + yn * yj + zn * zj
                            dx = xj - xn
                            dy = yj - yn
                            dz = zj - zn
                            coef[0, pl.ds(grp * L, L)] = _sqrt16(
                                dx * dx + dy * dy + dz * dz)
                            coef[1, pl.ds(grp * L, L)] = dot
                            coef[2, pl.ds(grp * L, L)] = dot / (rn * rj + EPS)
                        av = [abuf[cc, j, pl.ds(v * L, L)] for v in range(nv)]

                        def edge_body(k, acc, _p=p, _koff=koff):
                            crr = coef[0, pl.ds(k, L)][0]
                            cdot = coef[1, pl.ds(k, L)][0]
                            ccos = coef[2, pl.ds(k, L)][0]
                            out = []
                            for v in range(nv):
                                gv = gbuf[_p, _koff + k, pl.ds(v * L, L)]
                                pre = (av[v] + gv + crr * wrr[v]
                                       + cdot * wdot[v] + ccos * wcos[v])
                                out.append(acc[v] + _silu16(pre))
                            return tuple(out)

                        acc = lax.fori_loop(
                            0, k_nb, edge_body,
                            tuple(jnp.zeros((L,), jnp.float32)
                                  for _ in range(nv)),
                            unroll=2)
                        for v in range(nv):
                            pchunk[j, pl.ds(v * L, L)] = acc[v] * inv_k
                pltpu.sync_copy(pchunk, p_hbm.at[pl.ds(base + i0, CHUNK)])

        pltpu.make_async_copy(g_hbm.at[idx_v.at[pl.ds(0, NB * k_nb)]],
                              gbuf.at[0], sems[0]).wait()
        pltpu.make_async_copy(a_hbm.at[pl.ds(0, CHUNK)],
                              abuf.at[0], asems[0]).wait()

    return edge_kernel


def kernel(feat, xyz_centered, idx_knn, W1, W2):
    B, N, C = feat.shape
    K = idx_knn.shape[-1]
    assert B == 1 and C % L == 0 and K % L == 0
    npw = -(-N // NW)
    npw = -(-npw // 8) * 8
    n_pad = npw * NW
    pad = n_pad - N

    f_p = jnp.pad(feat[0], ((0, pad), (0, 0)))
    xyz_p = jnp.pad(xyz_centered[0], ((0, pad), (0, 0)))
    idx_p = jnp.pad(idx_knn[0], ((0, pad), (0, 0)))
    wc = W1[:C]
    wd = W1[C:2 * C]
    wrow = W1[2 * C:]

    a_t, g_t, r_t = pl.pallas_call(
        _prep_body,
        out_shape=[
            jax.ShapeDtypeStruct((n_pad, C), jnp.float32),
            jax.ShapeDtypeStruct((n_pad, C), jnp.float32),
            jax.ShapeDtypeStruct((n_pad, 1), jnp.float32),
        ],
    )(f_p, xyz_p, wc, wd, wrow)

    assert n_pad == NS * (NPW_F + NPW_S)
    idx_rows = (NS - 1) * (NPW_F + NPW_S) + 2 * NPW_F
    idx_flat = jnp.pad(idx_p.reshape(-1), (0, (idx_rows - n_pad) * K))
    edge = _make_edge_kernel(n_pad, K, C)
    p_t = edge(g_t, a_t, idx_flat,
               xyz_p[:, 0], xyz_p[:, 1], xyz_p[:, 2],
               r_t.reshape(n_pad),
               wrow[2], wrow[3], wrow[4])

    out = pl.pallas_call(
        _post_body,
        out_shape=jax.ShapeDtypeStruct((n_pad, C), jnp.float32),
    )(p_t, f_p, W2)
    return out[:N][None]
